# baseline (speedup 1.0000x reference)
import functools

import jax
import jax.numpy as jnp
from jax import lax
from jax.experimental import pallas as pl
from jax.experimental.pallas import tpu as pltpu
from jax.experimental.pallas import tpu_sc as plsc

_E = 64
_K = 2
_D = 1024
_F = 1024
_T = 2048
_A = _T * _K
_BLK = 128
_NT = _A // _BLK + _E
_PAD = _NT * _BLK
_CH = 64
_PC = 512


def _router_body(x_ref, gw_ref, e0_ref, e1_ref, p0_ref, p1_ref):
    lt = lax.dot_general(gw_ref[...], x_ref[...], (((1,), (1,)), ((), ())),
                         preferred_element_type=jnp.float32)
    a1 = jnp.argmax(lt, axis=0).astype(jnp.int32)
    m1 = jnp.max(lt, axis=0)
    ii = lax.broadcasted_iota(jnp.int32, lt.shape, 0)
    lt2 = jnp.where(ii == a1[None, :], -jnp.inf, lt)
    a2 = jnp.argmax(lt2, axis=0).astype(jnp.int32)
    m2 = jnp.max(lt2, axis=0)
    p = jax.nn.sigmoid(m1 - m2)
    e0_ref[...] = jnp.broadcast_to(a1[None, :], e0_ref.shape)
    e1_ref[...] = jnp.broadcast_to(a2[None, :], e1_ref.shape)
    p0_ref[...] = jnp.broadcast_to(p[None, :], p0_ref.shape)
    p1_ref[...] = jnp.broadcast_to((1.0 - p)[None, :], p1_ref.shape)


def _route(x, gate_w):
    return pl.pallas_call(
        _router_body,
        out_shape=[
            jax.ShapeDtypeStruct((8, _T), jnp.int32),
            jax.ShapeDtypeStruct((8, _T), jnp.int32),
            jax.ShapeDtypeStruct((8, _T), jnp.float32),
            jax.ShapeDtypeStruct((8, _T), jnp.float32),
        ],
    )(x, gate_w)


def _route_meta(e0, e1):
    ii = jnp.arange(_E, dtype=jnp.int32)[None, :]
    oh0 = (e0[:, None] == ii).astype(jnp.float32)
    oh1 = (e1[:, None] == ii).astype(jnp.float32)
    ohb = oh0 + oh1
    ci = jnp.arange(_PC, dtype=jnp.int32)
    tril = (ci[:, None] > ci[None, :]).astype(jnp.float32)
    ohc = ohb.reshape(_T // _PC, _PC, _E)
    within = jnp.einsum("ij,cjk->cik", tril, ohc,
                        preferred_element_type=jnp.float32)
    chunk_tot = jnp.cumsum(jnp.sum(ohc, axis=1), axis=0)
    carry = jnp.concatenate(
        [jnp.zeros((1, _E), jnp.float32), chunk_tot[:-1]], axis=0)
    prefix = (within + carry[:, None, :]).reshape(_T, _E)
    counts = chunk_tot[-1]
    tiles = jnp.ceil(counts / _BLK)
    ends = jnp.cumsum(tiles)
    starts = ends - tiles
    ntile = ends[-1].astype(jnp.int32)
    base = prefix + starts[None, :] * _BLK
    pos0 = jnp.sum(base * oh0, axis=1).astype(jnp.int32)
    pos1 = jnp.sum(base * oh1, axis=1).astype(jnp.int32)
    tt = jnp.arange(_NT, dtype=jnp.float32)
    tcl = jnp.minimum(tt, ends[-1] - 1.0)
    texp = jnp.sum((ends[None, :] <= tcl[:, None]).astype(jnp.int32),
                   axis=1).astype(jnp.int32)
    return pos0, pos1, texp, ntile.reshape(1)


_NC = 2
_NS = 16
_NW = _NC * _NS


def _sc_mesh():
    return plsc.VectorSubcoreMesh(core_axis_name="c", subcore_axis_name="s",
                                  num_cores=_NC, num_subcores=_NS)


@functools.lru_cache(maxsize=None)
def _make_sc_dispatch():
    per_w = _T // _NW

    @functools.partial(
        pl.kernel,
        mesh=_sc_mesh(),
        out_type=jax.ShapeDtypeStruct((_PAD, _D), jnp.float32),
        scratch_types=[
            pltpu.VMEM((per_w,), jnp.int32),
            pltpu.VMEM((per_w,), jnp.int32),
            pltpu.VMEM((per_w, _D), jnp.float32),
            pltpu.SemaphoreType.DMA,
        ],
    )
    def d(x_hbm, p0_hbm, p1_hbm, xs_hbm, i0_v, i1_v, buf_v, sem):
        wid = lax.axis_index("s") * _NC + lax.axis_index("c")
        base = wid * per_w
        a0 = pltpu.async_copy(p0_hbm.at[pl.ds(base, per_w)], i0_v, sem)
        a1 = pltpu.async_copy(p1_hbm.at[pl.ds(base, per_w)], i1_v, sem)
        a2 = pltpu.async_copy(x_hbm.at[pl.ds(base, per_w)], buf_v, sem)
        a0.wait()
        a1.wait()
        a2.wait()
        c0 = pltpu.async_copy(buf_v, xs_hbm.at[i0_v], sem)
        c1 = pltpu.async_copy(buf_v, xs_hbm.at[i1_v], sem)
        c0.wait()
        c1.wait()

    return d


@functools.lru_cache(maxsize=None)
def _make_sc_combine():
    tok_w = _T // _NW
    asn_w = tok_w * _K
    ch_t = 16
    ch_a = ch_t * _K
    nch = tok_w // ch_t
    lanes = _D // 16

    @functools.partial(
        pl.kernel,
        mesh=_sc_mesh(),
        out_type=jax.ShapeDtypeStruct((_T, _D), jnp.float32),
        scratch_types=[
            pltpu.VMEM((asn_w,), jnp.int32),
            pltpu.VMEM((asn_w, 16), jnp.float32),
            pltpu.VMEM((2, ch_a, _D), jnp.float32),
            pltpu.VMEM((2, ch_t, _D), jnp.float32),
            pltpu.SemaphoreType.DMA,
            pltpu.SemaphoreType.DMA,
            pltpu.SemaphoreType.DMA,
        ],
    )
    def g(y_hbm, pint_hbm, wv_hbm, out_hbm, idx_v, wv_v, buf_v, obuf_v,
          isem, gsem, wsem):
        wid = lax.axis_index("s") * _NC + lax.axis_index("c")
        abase = wid * asn_w
        tbase = wid * tok_w
        i0 = pltpu.async_copy(pint_hbm.at[pl.ds(abase, asn_w)], idx_v, isem)
        i1 = pltpu.async_copy(wv_hbm.at[pl.ds(abase, asn_w)], wv_v, isem)
        i0.wait()
        i1.wait()
        gathers = [None, None]
        writes = [None, None]
        gathers[0] = pltpu.async_copy(
            y_hbm.at[idx_v.at[pl.ds(0, ch_a)]], buf_v.at[0], gsem)
        for c in range(nch):
            par = c % 2
            gathers[par].wait()
            if writes[par] is not None:
                writes[par].wait()
            if c + 1 < nch:
                gathers[1 - par] = pltpu.async_copy(
                    y_hbm.at[idx_v.at[pl.ds((c + 1) * ch_a, ch_a)]],
                    buf_v.at[1 - par], gsem)

            def tok(k, carry):
                w0 = wv_v[c * ch_a + 2 * k]
                w1 = wv_v[c * ch_a + 2 * k + 1]

                def lane(j, carry2):
                    for u in range(4):
                        s = pl.ds((j * 4 + u) * 16, 16)
                        obuf_v[par, k, s] = (w0 * buf_v[par, 2 * k, s]
                                             + w1 * buf_v[par, 2 * k + 1, s])
                    return carry2

                lax.fori_loop(0, lanes // 4, lane, 0)
                return carry

            lax.fori_loop(0, ch_t, tok, 0)
            writes[par] = pltpu.async_copy(
                obuf_v.at[par], out_hbm.at[pl.ds(tbase + c * ch_t, ch_t)],
                wsem)
        writes[0].wait()
        writes[1].wait()

    return g


def _ffn_body(texp_ref, nt_ref, x_ref, w1_ref, w3_ref, w2_ref, y_ref):
    t = pl.program_id(0)

    @pl.when(t < nt_ref[0])
    def _():
        x = x_ref[...]
        h1 = jnp.dot(x, w1_ref[0], preferred_element_type=jnp.float32)
        h3 = jnp.dot(x, w3_ref[0], preferred_element_type=jnp.float32)
        h = (h1 * jax.nn.sigmoid(h1)) * h3
        y_ref[...] = jnp.dot(h, w2_ref[0], preferred_element_type=jnp.float32)


def _ffn(texp, nt, xs, w1, w3, w2):
    grid_spec = pltpu.PrefetchScalarGridSpec(
        num_scalar_prefetch=2,
        grid=(_NT,),
        in_specs=[
            pl.BlockSpec((_BLK, _D),
                         lambda t, texp, nt: (jnp.minimum(t, nt[0] - 1), 0)),
            pl.BlockSpec((1, _D, _F), lambda t, texp, nt: (texp[t], 0, 0)),
            pl.BlockSpec((1, _D, _F), lambda t, texp, nt: (texp[t], 0, 0)),
            pl.BlockSpec((1, _F, _D), lambda t, texp, nt: (texp[t], 0, 0)),
        ],
        out_specs=pl.BlockSpec((_BLK, _D),
                               lambda t, texp, nt: (jnp.minimum(t, nt[0] - 1), 0)),
    )
    return pl.pallas_call(
        _ffn_body,
        grid_spec=grid_spec,
        out_shape=jax.ShapeDtypeStruct((_PAD, _D), jnp.float32),
    )(texp, nt, xs, w1, w3, w2)


def kernel(hidden_states, gate_w, w1, w3, w2):
    orig_shape = hidden_states.shape
    x = hidden_states.reshape(_T, _D)
    e0_2d, e1_2d, p0_2d, p1_2d = _route(x, gate_w)
    pos0, pos1, texp, nt = _route_meta(e0_2d[0], e1_2d[0])
    xs = _make_sc_dispatch()(x, pos0, pos1)
    y = _ffn(texp, nt, xs, w1, w3, w2)
    pint = jnp.stack([pos0, pos1], axis=1).reshape(-1)
    wint = jnp.stack([p0_2d[0], p1_2d[0]], axis=1).reshape(-1)
    wv = jnp.broadcast_to(wint[:, None], (_A, 16))
    out = _make_sc_combine()(y, pint, wv)
    return out.reshape(orig_shape)


# speedup vs baseline: 1.0011x; 1.0011x over previous
import functools

import jax
import jax.numpy as jnp
from jax import lax
from jax.experimental import pallas as pl
from jax.experimental.pallas import tpu as pltpu
from jax.experimental.pallas import tpu_sc as plsc

_E = 64
_K = 2
_D = 1024
_F = 1024
_T = 2048
_A = _T * _K
_BLK = 128
_NT = _A // _BLK + _E
_PAD = _NT * _BLK
_CH = 64
_PC = 512


def _router_body(x_ref, gw_ref, e0_ref, e1_ref, p0_ref, p1_ref):
    lt = lax.dot_general(gw_ref[...], x_ref[...], (((1,), (1,)), ((), ())),
                         preferred_element_type=jnp.float32)
    a1 = jnp.argmax(lt, axis=0).astype(jnp.int32)
    m1 = jnp.max(lt, axis=0)
    ii = lax.broadcasted_iota(jnp.int32, lt.shape, 0)
    lt2 = jnp.where(ii == a1[None, :], -jnp.inf, lt)
    a2 = jnp.argmax(lt2, axis=0).astype(jnp.int32)
    m2 = jnp.max(lt2, axis=0)
    p = jax.nn.sigmoid(m1 - m2)
    e0_ref[...] = jnp.broadcast_to(a1[None, :], e0_ref.shape)
    e1_ref[...] = jnp.broadcast_to(a2[None, :], e1_ref.shape)
    p0_ref[...] = jnp.broadcast_to(p[None, :], p0_ref.shape)
    p1_ref[...] = jnp.broadcast_to((1.0 - p)[None, :], p1_ref.shape)


def _route(x, gate_w):
    return pl.pallas_call(
        _router_body,
        out_shape=[
            jax.ShapeDtypeStruct((8, _T), jnp.int32),
            jax.ShapeDtypeStruct((8, _T), jnp.int32),
            jax.ShapeDtypeStruct((8, _T), jnp.float32),
            jax.ShapeDtypeStruct((8, _T), jnp.float32),
        ],
    )(x, gate_w)


def _route_meta(e0, e1):
    ii = jnp.arange(_E, dtype=jnp.int32)[None, :]
    oh0 = (e0[:, None] == ii).astype(jnp.float32)
    oh1 = (e1[:, None] == ii).astype(jnp.float32)
    ohb = oh0 + oh1
    ci = jnp.arange(_PC, dtype=jnp.int32)
    tril = (ci[:, None] > ci[None, :]).astype(jnp.float32)
    ohc = ohb.reshape(_T // _PC, _PC, _E)
    within = jnp.einsum("ij,cjk->cik", tril, ohc,
                        preferred_element_type=jnp.float32)
    chunk_tot = jnp.cumsum(jnp.sum(ohc, axis=1), axis=0)
    carry = jnp.concatenate(
        [jnp.zeros((1, _E), jnp.float32), chunk_tot[:-1]], axis=0)
    prefix = (within + carry[:, None, :]).reshape(_T, _E)
    counts = chunk_tot[-1]
    tiles = jnp.ceil(counts / _BLK)
    ends = jnp.cumsum(tiles)
    starts = ends - tiles
    ntile = ends[-1].astype(jnp.int32)
    base = prefix + starts[None, :] * _BLK
    pos0 = jnp.sum(base * oh0, axis=1).astype(jnp.int32)
    pos1 = jnp.sum(base * oh1, axis=1).astype(jnp.int32)
    tt = jnp.arange(_NT, dtype=jnp.float32)
    tcl = jnp.minimum(tt, ends[-1] - 1.0)
    texp = jnp.sum((ends[None, :] <= tcl[:, None]).astype(jnp.int32),
                   axis=1).astype(jnp.int32)
    return pos0, pos1, texp, ntile.reshape(1)


_NC = 2
_NS = 16
_NW = _NC * _NS


def _sc_mesh():
    return plsc.VectorSubcoreMesh(core_axis_name="c", subcore_axis_name="s",
                                  num_cores=_NC, num_subcores=_NS)


@functools.lru_cache(maxsize=None)
def _make_sc_dispatch():
    per_w = _T // _NW

    @functools.partial(
        pl.kernel,
        mesh=_sc_mesh(),
        out_type=jax.ShapeDtypeStruct((_PAD, _D), jnp.float32),
        scratch_types=[
            pltpu.VMEM((per_w,), jnp.int32),
            pltpu.VMEM((per_w,), jnp.int32),
            pltpu.VMEM((per_w, _D), jnp.float32),
            pltpu.SemaphoreType.DMA,
        ],
    )
    def d(x_hbm, p0_hbm, p1_hbm, xs_hbm, i0_v, i1_v, buf_v, sem):
        wid = lax.axis_index("s") * _NC + lax.axis_index("c")
        base = wid * per_w
        a0 = pltpu.async_copy(p0_hbm.at[pl.ds(base, per_w)], i0_v, sem)
        a1 = pltpu.async_copy(p1_hbm.at[pl.ds(base, per_w)], i1_v, sem)
        a2 = pltpu.async_copy(x_hbm.at[pl.ds(base, per_w)], buf_v, sem)
        a0.wait()
        a1.wait()
        a2.wait()
        c0 = pltpu.async_copy(buf_v, xs_hbm.at[i0_v], sem)
        c1 = pltpu.async_copy(buf_v, xs_hbm.at[i1_v], sem)
        c0.wait()
        c1.wait()

    return d


@functools.lru_cache(maxsize=None)
def _make_sc_combine():
    tok_w = _T // _NW
    asn_w = tok_w * _K
    ch_t = 16
    ch_a = ch_t * _K
    nch = tok_w // ch_t
    lanes = _D // 16

    @functools.partial(
        pl.kernel,
        mesh=_sc_mesh(),
        out_type=jax.ShapeDtypeStruct((_T, _D), jnp.float32),
        scratch_types=[
            pltpu.VMEM((asn_w,), jnp.int32),
            pltpu.VMEM((asn_w, 16), jnp.float32),
            pltpu.VMEM((2, ch_a, _D), jnp.float32),
            pltpu.VMEM((2, ch_t, _D), jnp.float32),
            pltpu.SemaphoreType.DMA,
            pltpu.SemaphoreType.DMA,
            pltpu.SemaphoreType.DMA,
        ],
    )
    def g(y_hbm, pint_hbm, wv_hbm, out_hbm, idx_v, wv_v, buf_v, obuf_v,
          isem, gsem, wsem):
        wid = lax.axis_index("s") * _NC + lax.axis_index("c")
        abase = wid * asn_w
        tbase = wid * tok_w
        i0 = pltpu.async_copy(pint_hbm.at[pl.ds(abase, asn_w)], idx_v, isem)
        i1 = pltpu.async_copy(wv_hbm.at[pl.ds(abase, asn_w)], wv_v, isem)
        i0.wait()
        i1.wait()
        gathers = [None, None]
        writes = [None, None]
        gathers[0] = pltpu.async_copy(
            y_hbm.at[idx_v.at[pl.ds(0, ch_a)]], buf_v.at[0], gsem)
        for c in range(nch):
            par = c % 2
            gathers[par].wait()
            if writes[par] is not None:
                writes[par].wait()
            if c + 1 < nch:
                gathers[1 - par] = pltpu.async_copy(
                    y_hbm.at[idx_v.at[pl.ds((c + 1) * ch_a, ch_a)]],
                    buf_v.at[1 - par], gsem)

            def tok(k, carry):
                w0 = wv_v[c * ch_a + 2 * k]
                w1 = wv_v[c * ch_a + 2 * k + 1]

                def lane(j, carry2):
                    s = pl.ds(j * 16, 16)
                    obuf_v[par, k, s] = (w0 * buf_v[par, 2 * k, s]
                                         + w1 * buf_v[par, 2 * k + 1, s])
                    return carry2

                lax.fori_loop(0, lanes, lane, 0)
                return carry

            lax.fori_loop(0, ch_t, tok, 0)
            writes[par] = pltpu.async_copy(
                obuf_v.at[par], out_hbm.at[pl.ds(tbase + c * ch_t, ch_t)],
                wsem)
        writes[0].wait()
        writes[1].wait()

    return g


def _ffn_body(texp_ref, nt_ref, x_ref, w1_ref, w3_ref, w2_ref, y_ref):
    t = pl.program_id(0)

    @pl.when(t < nt_ref[0])
    def _():
        x = x_ref[...]
        h1 = jnp.dot(x, w1_ref[0], preferred_element_type=jnp.float32)
        h3 = jnp.dot(x, w3_ref[0], preferred_element_type=jnp.float32)
        h = (h1 * jax.nn.sigmoid(h1)) * h3
        y_ref[...] = jnp.dot(h, w2_ref[0], preferred_element_type=jnp.float32)


def _ffn(texp, nt, xs, w1, w3, w2):
    grid_spec = pltpu.PrefetchScalarGridSpec(
        num_scalar_prefetch=2,
        grid=(_NT,),
        in_specs=[
            pl.BlockSpec((_BLK, _D),
                         lambda t, texp, nt: (jnp.minimum(t, nt[0] - 1), 0)),
            pl.BlockSpec((1, _D, _F), lambda t, texp, nt: (texp[t], 0, 0)),
            pl.BlockSpec((1, _D, _F), lambda t, texp, nt: (texp[t], 0, 0)),
            pl.BlockSpec((1, _F, _D), lambda t, texp, nt: (texp[t], 0, 0)),
        ],
        out_specs=pl.BlockSpec((_BLK, _D),
                               lambda t, texp, nt: (jnp.minimum(t, nt[0] - 1), 0)),
    )
    return pl.pallas_call(
        _ffn_body,
        grid_spec=grid_spec,
        out_shape=jax.ShapeDtypeStruct((_PAD, _D), jnp.float32),
    )(texp, nt, xs, w1, w3, w2)


def kernel(hidden_states, gate_w, w1, w3, w2):
    orig_shape = hidden_states.shape
    x = hidden_states.reshape(_T, _D)
    e0_2d, e1_2d, p0_2d, p1_2d = _route(x, gate_w)
    pos0, pos1, texp, nt = _route_meta(e0_2d[0], e1_2d[0])
    xs = _make_sc_dispatch()(x, pos0, pos1)
    y = _ffn(texp, nt, xs, w1, w3, w2)
    pint = jnp.stack([pos0, pos1], axis=1).reshape(-1)
    wint = jnp.stack([p0_2d[0], p1_2d[0]], axis=1).reshape(-1)
    wv = jnp.broadcast_to(wint[:, None], (_A, 16))
    out = _make_sc_combine()(y, pint, wv)
    return out.reshape(orig_shape)


# speedup vs baseline: 1.0574x; 1.0562x over previous
import functools

import jax
import jax.numpy as jnp
from jax import lax
from jax.experimental import pallas as pl
from jax.experimental.pallas import tpu as pltpu
from jax.experimental.pallas import tpu_sc as plsc

_E = 64
_K = 2
_D = 1024
_F = 1024
_T = 2048
_A = _T * _K
_BLK = 128
_NT = _A // _BLK + _E
_PAD = _NT * _BLK
_PC = 512


def _router_body(x_ref, gw_ref, e0_ref, e1_ref, p0_ref, p1_ref):
    lt = lax.dot_general(gw_ref[...], x_ref[...], (((1,), (1,)), ((), ())),
                         preferred_element_type=jnp.float32)
    a1 = jnp.argmax(lt, axis=0).astype(jnp.int32)
    m1 = jnp.max(lt, axis=0)
    ii = lax.broadcasted_iota(jnp.int32, lt.shape, 0)
    lt2 = jnp.where(ii == a1[None, :], -jnp.inf, lt)
    a2 = jnp.argmax(lt2, axis=0).astype(jnp.int32)
    m2 = jnp.max(lt2, axis=0)
    p = jax.nn.sigmoid(m1 - m2)
    e0_ref[...] = jnp.broadcast_to(a1[None, :], e0_ref.shape)
    e1_ref[...] = jnp.broadcast_to(a2[None, :], e1_ref.shape)
    p0_ref[...] = jnp.broadcast_to(p[None, :], p0_ref.shape)
    p1_ref[...] = jnp.broadcast_to((1.0 - p)[None, :], p1_ref.shape)


def _route(x, gate_w):
    return pl.pallas_call(
        _router_body,
        out_shape=[
            jax.ShapeDtypeStruct((8, _T), jnp.int32),
            jax.ShapeDtypeStruct((8, _T), jnp.int32),
            jax.ShapeDtypeStruct((8, _T), jnp.float32),
            jax.ShapeDtypeStruct((8, _T), jnp.float32),
        ],
    )(x, gate_w)


def _route_meta(e0, e1):
    ii = jnp.arange(_E, dtype=jnp.int32)[None, :]
    oh0 = (e0[:, None] == ii).astype(jnp.float32)
    oh1 = (e1[:, None] == ii).astype(jnp.float32)
    ohb = oh0 + oh1
    ci = jnp.arange(_PC, dtype=jnp.int32)
    tril = (ci[:, None] > ci[None, :]).astype(jnp.float32)
    ohc = ohb.reshape(_T // _PC, _PC, _E)
    within = jnp.einsum("ij,cjk->cik", tril, ohc,
                        preferred_element_type=jnp.float32)
    chunk_tot = jnp.cumsum(jnp.sum(ohc, axis=1), axis=0)
    carry = jnp.concatenate(
        [jnp.zeros((1, _E), jnp.float32), chunk_tot[:-1]], axis=0)
    prefix = (within + carry[:, None, :]).reshape(_T, _E)
    counts = chunk_tot[-1]
    tiles = jnp.ceil(counts / _BLK)
    ends = jnp.cumsum(tiles)
    starts = ends - tiles
    ntile = ends[-1].astype(jnp.int32)
    base = prefix + starts[None, :] * _BLK
    pos0 = jnp.sum(base * oh0, axis=1).astype(jnp.int32)
    pos1 = jnp.sum(base * oh1, axis=1).astype(jnp.int32)
    tt = jnp.arange(_NT, dtype=jnp.float32)
    tcl = jnp.minimum(tt, ends[-1] - 1.0)
    texp = jnp.sum((ends[None, :] <= tcl[:, None]).astype(jnp.int32),
                   axis=1).astype(jnp.int32)
    return pos0, pos1, texp, ntile.reshape(1)


_NC = 2
_NS = 16
_NW = _NC * _NS


def _sc_mesh():
    return plsc.VectorSubcoreMesh(core_axis_name="c", subcore_axis_name="s",
                                  num_cores=_NC, num_subcores=_NS)


@functools.lru_cache(maxsize=None)
def _make_sc_dispatch():
    per_w = _T // _NW

    @functools.partial(
        pl.kernel,
        mesh=_sc_mesh(),
        out_type=jax.ShapeDtypeStruct((_PAD, _D), jnp.float32),
        scratch_types=[
            pltpu.VMEM((per_w,), jnp.int32),
            pltpu.VMEM((per_w,), jnp.int32),
            pltpu.VMEM((per_w, _D), jnp.float32),
            pltpu.SemaphoreType.DMA,
        ],
    )
    def d(x_hbm, p0_hbm, p1_hbm, xs_hbm, i0_v, i1_v, buf_v, sem):
        wid = lax.axis_index("s") * _NC + lax.axis_index("c")
        base = wid * per_w
        a0 = pltpu.async_copy(p0_hbm.at[pl.ds(base, per_w)], i0_v, sem)
        a1 = pltpu.async_copy(p1_hbm.at[pl.ds(base, per_w)], i1_v, sem)
        a2 = pltpu.async_copy(x_hbm.at[pl.ds(base, per_w)], buf_v, sem)
        a0.wait()
        a1.wait()
        a2.wait()
        c0 = pltpu.async_copy(buf_v, xs_hbm.at[i0_v], sem)
        c1 = pltpu.async_copy(buf_v, xs_hbm.at[i1_v], sem)
        c0.wait()
        c1.wait()

    return d


@functools.lru_cache(maxsize=None)
def _make_sc_combine():
    tok_w = _T // _NW
    asn_w = tok_w * _K
    ch_t = 16
    ch_a = ch_t * _K
    nch = tok_w // ch_t
    lanes = _D // 16

    @functools.partial(
        pl.kernel,
        mesh=_sc_mesh(),
        out_type=jax.ShapeDtypeStruct((_T, _D), jnp.float32),
        scratch_types=[
            pltpu.VMEM((asn_w,), jnp.int32),
            pltpu.VMEM((asn_w, 16), jnp.float32),
            pltpu.VMEM((2, ch_a, _D), jnp.float32),
            pltpu.VMEM((2, ch_t, _D), jnp.float32),
            pltpu.SemaphoreType.DMA,
            pltpu.SemaphoreType.DMA,
            pltpu.SemaphoreType.DMA,
        ],
    )
    def g(y_hbm, pint_hbm, wv_hbm, out_hbm, idx_v, wv_v, buf_v, obuf_v,
          isem, gsem, wsem):
        wid = lax.axis_index("s") * _NC + lax.axis_index("c")
        abase = wid * asn_w
        tbase = wid * tok_w
        i0 = pltpu.async_copy(pint_hbm.at[pl.ds(abase, asn_w)], idx_v, isem)
        i1 = pltpu.async_copy(wv_hbm.at[pl.ds(abase, asn_w)], wv_v, isem)
        i0.wait()
        i1.wait()
        gathers = [None, None]
        writes = [None, None]
        gathers[0] = pltpu.async_copy(
            y_hbm.at[idx_v.at[pl.ds(0, ch_a)]], buf_v.at[0], gsem)
        for c in range(nch):
            par = c % 2
            gathers[par].wait()
            if writes[par] is not None:
                writes[par].wait()
            if c + 1 < nch:
                gathers[1 - par] = pltpu.async_copy(
                    y_hbm.at[idx_v.at[pl.ds((c + 1) * ch_a, ch_a)]],
                    buf_v.at[1 - par], gsem)

            @plsc.parallel_loop(0, ch_t)
            def _tok(k):
                w0 = wv_v[c * ch_a + 2 * k]
                w1 = wv_v[c * ch_a + 2 * k + 1]

                @plsc.parallel_loop(0, lanes, unroll=4)
                def _lane(j):
                    s = pl.ds(j * 16, 16)
                    obuf_v[par, k, s] = (w0 * buf_v[par, 2 * k, s]
                                         + w1 * buf_v[par, 2 * k + 1, s])
            writes[par] = pltpu.async_copy(
                obuf_v.at[par], out_hbm.at[pl.ds(tbase + c * ch_t, ch_t)],
                wsem)
        writes[0].wait()
        writes[1].wait()

    return g


def _ffn_body(texp_ref, nt_ref, x_ref, w1_ref, w3_ref, w2_ref, y_ref):
    t = pl.program_id(0)

    @pl.when(t < nt_ref[0])
    def _():
        x = x_ref[...]
        h1 = jnp.dot(x, w1_ref[0], preferred_element_type=jnp.float32)
        h3 = jnp.dot(x, w3_ref[0], preferred_element_type=jnp.float32)
        h = (h1 * jax.nn.sigmoid(h1)) * h3
        y_ref[...] = jnp.dot(h, w2_ref[0], preferred_element_type=jnp.float32)


def _ffn(texp, nt, xs, w1, w3, w2):
    grid_spec = pltpu.PrefetchScalarGridSpec(
        num_scalar_prefetch=2,
        grid=(_NT,),
        in_specs=[
            pl.BlockSpec((_BLK, _D),
                         lambda t, texp, nt: (jnp.minimum(t, nt[0] - 1), 0)),
            pl.BlockSpec((1, _D, _F), lambda t, texp, nt: (texp[t], 0, 0)),
            pl.BlockSpec((1, _D, _F), lambda t, texp, nt: (texp[t], 0, 0)),
            pl.BlockSpec((1, _F, _D), lambda t, texp, nt: (texp[t], 0, 0)),
        ],
        out_specs=pl.BlockSpec((_BLK, _D),
                               lambda t, texp, nt: (jnp.minimum(t, nt[0] - 1), 0)),
    )
    return pl.pallas_call(
        _ffn_body,
        grid_spec=grid_spec,
        out_shape=jax.ShapeDtypeStruct((_PAD, _D), jnp.float32),
    )(texp, nt, xs, w1, w3, w2)


def kernel(hidden_states, gate_w, w1, w3, w2):
    orig_shape = hidden_states.shape
    x = hidden_states.reshape(_T, _D)
    e0_2d, e1_2d, p0_2d, p1_2d = _route(x, gate_w)
    pos0, pos1, texp, nt = _route_meta(e0_2d[0], e1_2d[0])
    xs = _make_sc_dispatch()(x, pos0, pos1)
    y = _ffn(texp, nt, xs, w1, w3, w2)
    pint = jnp.stack([pos0, pos1], axis=1).reshape(-1)
    wint = jnp.stack([p0_2d[0], p1_2d[0]], axis=1).reshape(-1)
    wv = jnp.broadcast_to(wint[:, None], (_A, 16))
    out = _make_sc_combine()(y, pint, wv)
    return out.reshape(orig_shape)
